# baseline (speedup 1.0000x reference)
import functools

import jax
import jax.numpy as jnp
from jax import lax
from jax.experimental import pallas as pl
from jax.experimental.pallas import tpu as pltpu
from jax.experimental.pallas import tpu_sc as plsc

_BUCKET_NUM = 100
_BUCKET_START = 0
_FEATURE_DIM = 128
_START_SMOOTH = 1
_MIN_VALUE = 0.0
_BIN_WIDTH = 1.0 / (_BUCKET_NUM - 1)
_N = 131072

_NBUCKETS = _BUCKET_NUM - _BUCKET_START
_LANE = 16
_NGROUPS = _FEATURE_DIM // _LANE


def _prep_body(m1_ref, v1_ref, m2_ref, v2_ref, epoch_ref, comb_ref):
    m1 = m1_ref[...]
    v1 = v1_ref[...]
    m2 = m2_ref[...]
    v2 = v2_ref[...]
    factor = jnp.clip(v2 / jnp.maximum(v1, 1e-12), 0.1, 10.0)
    s = jnp.sqrt(factor)
    ok = v1 > 1e-12
    scale = jnp.where(ok, s, 1.0)
    bias = jnp.where(ok, m2 - m1 * scale, 0.0)
    smooth = epoch_ref[0] >= _START_SMOOTH
    scale = jnp.where(smooth, scale, jnp.ones_like(scale))
    bias = jnp.where(smooth, bias, jnp.zeros_like(bias))
    s16 = lax.bitcast_convert_type(
        scale.astype(jnp.bfloat16), jnp.uint16).astype(jnp.uint32)
    t16 = lax.bitcast_convert_type(
        bias.astype(jnp.bfloat16), jnp.uint16).astype(jnp.uint32)
    comb_ref[...] = lax.bitcast_convert_type(
        (t16 << 16) | s16, jnp.int32)


def _prep_tables(m1, v1, m2, v2, epoch_arr):
    return pl.pallas_call(
        _prep_body,
        out_shape=jax.ShapeDtypeStruct((_NBUCKETS, _FEATURE_DIM), jnp.int32),
        in_specs=[
            pl.BlockSpec(memory_space=pltpu.VMEM),
            pl.BlockSpec(memory_space=pltpu.VMEM),
            pl.BlockSpec(memory_space=pltpu.VMEM),
            pl.BlockSpec(memory_space=pltpu.VMEM),
            pl.BlockSpec(memory_space=pltpu.SMEM),
        ],
        out_specs=pl.BlockSpec(memory_space=pltpu.VMEM),
    )(m1, v1, m2, v2, epoch_arr)


def _make_sc_kernel():
    info = plsc.get_sparse_core_info()
    nc, ns = info.num_cores, info.num_subcores
    nw = nc * ns
    rows_per_w = _N // nw
    chunk = 128
    nchunks = rows_per_w // chunk
    nbuf_in = 4
    nbuf_out = 2

    mesh = plsc.VectorSubcoreMesh(core_axis_name="c", subcore_axis_name="s")

    @functools.partial(
        pl.kernel,
        mesh=mesh,
        out_type=jax.ShapeDtypeStruct((_N, _FEATURE_DIM), jnp.float32),
        scratch_types=[
            pltpu.VMEM((_NBUCKETS * _FEATURE_DIM,), jnp.int32),
            pltpu.VMEM((rows_per_w,), jnp.float32),
            pltpu.VMEM((rows_per_w + _LANE,), jnp.int32),
            pltpu.VMEM((nbuf_in, chunk, _FEATURE_DIM), jnp.float32),
            pltpu.VMEM((nbuf_out, chunk, _FEATURE_DIM), jnp.float32),
            pltpu.SemaphoreType.DMA,
            pltpu.SemaphoreType.DMA,
            pltpu.SemaphoreType.DMA,
            pltpu.SemaphoreType.DMA,
            pltpu.SemaphoreType.DMA,
            pltpu.SemaphoreType.DMA,
        ],
    )
    def sc_kernel(features_hbm, labels_hbm, comb_hbm, out_hbm,
                  comb_v, lab_v, bins_v, in_v, out_v,
                  sem_in0, sem_in1, sem_in2, sem_in3, sem_out0, sem_out1):
        sem_in = (sem_in0, sem_in1, sem_in2, sem_in3)
        sem_out = (sem_out0, sem_out1)
        wid = lax.axis_index("s") * nc + lax.axis_index("c")
        base = wid * rows_per_w

        pltpu.sync_copy(comb_hbm, comb_v)
        pltpu.sync_copy(labels_hbm.at[pl.ds(base, rows_per_w)], lab_v)

        def binify(k, _):
            lv = lab_v[pl.ds(k * _LANE, _LANE)]
            b = ((lv - _MIN_VALUE) * (1.0 / _BIN_WIDTH)).astype(jnp.int32)
            b = jnp.minimum(b, _NBUCKETS - 1) << 7
            bins_v[pl.ds(k * _LANE, _LANE)] = b
            return _

        lax.fori_loop(0, rows_per_w // _LANE, binify, 0)

        for b in range(nbuf_in):
            pltpu.async_copy(
                features_hbm.at[pl.ds(base + b * chunk, chunk)],
                in_v.at[b], sem_in[b])

        hi_mask = jnp.int32(-65536)

        def do_chunk(c, bi, bo):
            row0 = base + c * chunk
            pltpu.make_async_copy(
                features_hbm.at[pl.ds(row0, chunk)], in_v.at[bi],
                sem_in[bi]).wait()
            @pl.when(c >= nbuf_out)
            def _():
                pltpu.make_async_copy(
                    out_v.at[bo], out_hbm.at[pl.ds(row0, chunk)],
                    sem_out[bo]).wait()

            def row_body(g, bvec):
                bvec_n = bins_v[pl.ds(c * chunk + (g + 1) * _LANE, _LANE)]
                for l0 in range(0, _LANE, 2):
                    vals = []
                    for l in (l0, l0 + 1):
                        i = g * _LANE + l
                        boff = bvec[l]
                        for j in range(_NGROUPS):
                            x = in_v[bi, i, pl.ds(j * _LANE, _LANE)]
                            w = comb_v[pl.ds(boff + j * _LANE, _LANE)]
                            s = lax.bitcast_convert_type(w << 16, jnp.float32)
                            t = lax.bitcast_convert_type(w & hi_mask,
                                                         jnp.float32)
                            vals.append((i, j, x * s + t))
                    for i, j, r in vals:
                        out_v[bo, i, pl.ds(j * _LANE, _LANE)] = r
                return bvec_n

            bvec0 = bins_v[pl.ds(c * chunk, _LANE)]
            lax.fori_loop(0, chunk // _LANE, row_body, bvec0)

            pltpu.async_copy(
                out_v.at[bo], out_hbm.at[pl.ds(row0, chunk)], sem_out[bo])

            @pl.when(c + nbuf_in < nchunks)
            def _():
                pltpu.async_copy(
                    features_hbm.at[pl.ds(row0 + nbuf_in * chunk, chunk)],
                    in_v.at[bi], sem_in[bi])

        def outer(cc, _):
            for b in range(nbuf_in):
                c = cc * nbuf_in + b
                do_chunk(c, b, b % nbuf_out)
            return _

        lax.fori_loop(0, nchunks // nbuf_in, outer, 0)

        for b in range(nbuf_out):
            c = nchunks - nbuf_out + b
            pltpu.make_async_copy(
                out_v.at[(nchunks - nbuf_out + b) % nbuf_out],
                out_hbm.at[pl.ds(base + c * chunk, chunk)],
                sem_out[(nchunks - nbuf_out + b) % nbuf_out]).wait()

    return sc_kernel


_sc_kernel = None


def kernel(features, labels, running_mean_last_epoch, running_var_last_epoch,
           smoothed_mean_last_epoch, smoothed_var_last_epoch, epoch):
    global _sc_kernel
    if _sc_kernel is None:
        _sc_kernel = _make_sc_kernel()
    epoch_arr = jnp.asarray(epoch, dtype=jnp.int32).reshape((1,))
    comb = _prep_tables(
        running_mean_last_epoch, running_var_last_epoch,
        smoothed_mean_last_epoch, smoothed_var_last_epoch, epoch_arr)
    labels_flat = labels.reshape((_N,))
    return _sc_kernel(features, labels_flat, comb.reshape((-1,)))


# speedup vs baseline: 1.0381x; 1.0381x over previous
import functools

import jax
import jax.numpy as jnp
from jax import lax
from jax.experimental import pallas as pl
from jax.experimental.pallas import tpu as pltpu
from jax.experimental.pallas import tpu_sc as plsc

_BUCKET_NUM = 100
_BUCKET_START = 0
_FEATURE_DIM = 128
_START_SMOOTH = 1
_MIN_VALUE = 0.0
_BIN_WIDTH = 1.0 / (_BUCKET_NUM - 1)
_N = 131072

_NBUCKETS = _BUCKET_NUM - _BUCKET_START
_LANE = 16
_NGROUPS = _FEATURE_DIM // _LANE


def _prep_body(m1_ref, v1_ref, m2_ref, v2_ref, epoch_ref, comb_ref):
    m1 = m1_ref[...]
    v1 = v1_ref[...]
    m2 = m2_ref[...]
    v2 = v2_ref[...]
    factor = jnp.clip(v2 / jnp.maximum(v1, 1e-12), 0.1, 10.0)
    s = jnp.sqrt(factor)
    ok = v1 > 1e-12
    scale = jnp.where(ok, s, 1.0)
    bias = jnp.where(ok, m2 - m1 * scale, 0.0)
    smooth = epoch_ref[0] >= _START_SMOOTH
    scale = jnp.where(smooth, scale, jnp.ones_like(scale))
    bias = jnp.where(smooth, bias, jnp.zeros_like(bias))
    s16 = lax.bitcast_convert_type(
        scale.astype(jnp.bfloat16), jnp.uint16).astype(jnp.uint32)
    t16 = lax.bitcast_convert_type(
        bias.astype(jnp.bfloat16), jnp.uint16).astype(jnp.uint32)
    comb_ref[...] = lax.bitcast_convert_type(
        (t16 << 16) | s16, jnp.int32)


def _prep_tables(m1, v1, m2, v2, epoch_arr):
    return pl.pallas_call(
        _prep_body,
        out_shape=jax.ShapeDtypeStruct((_NBUCKETS, _FEATURE_DIM), jnp.int32),
        in_specs=[
            pl.BlockSpec(memory_space=pltpu.VMEM),
            pl.BlockSpec(memory_space=pltpu.VMEM),
            pl.BlockSpec(memory_space=pltpu.VMEM),
            pl.BlockSpec(memory_space=pltpu.VMEM),
            pl.BlockSpec(memory_space=pltpu.SMEM),
        ],
        out_specs=pl.BlockSpec(memory_space=pltpu.VMEM),
    )(m1, v1, m2, v2, epoch_arr)


def _make_sc_kernel():
    info = plsc.get_sparse_core_info()
    nc, ns = info.num_cores, info.num_subcores
    nw = nc * ns
    rows_per_w = _N // nw
    chunk = 128
    nchunks = rows_per_w // chunk
    nbuf_in = 4
    nbuf_out = 2

    mesh = plsc.VectorSubcoreMesh(core_axis_name="c", subcore_axis_name="s")

    @functools.partial(
        pl.kernel,
        mesh=mesh,
        out_type=jax.ShapeDtypeStruct((_N, _FEATURE_DIM), jnp.float32),
        scratch_types=[
            pltpu.VMEM((_NBUCKETS * _FEATURE_DIM,), jnp.int32),
            pltpu.VMEM((rows_per_w,), jnp.float32),
            pltpu.VMEM((rows_per_w + _LANE,), jnp.int32),
            pltpu.VMEM((nbuf_in, chunk, _FEATURE_DIM), jnp.float32),
            pltpu.VMEM((nbuf_out, chunk, _FEATURE_DIM), jnp.float32),
            pltpu.SemaphoreType.DMA,
            pltpu.SemaphoreType.DMA,
            pltpu.SemaphoreType.DMA,
            pltpu.SemaphoreType.DMA,
            pltpu.SemaphoreType.DMA,
            pltpu.SemaphoreType.DMA,
            pltpu.SemaphoreType.DMA,
        ],
    )
    def sc_kernel(features_hbm, labels_hbm, comb_hbm, out_hbm,
                  comb_v, lab_v, bins_v, in_v, out_v,
                  sem_in0, sem_in1, sem_in2, sem_in3, sem_out0, sem_out1,
                  sem_tab):
        sem_in = (sem_in0, sem_in1, sem_in2, sem_in3)
        sem_out = (sem_out0, sem_out1)
        wid = lax.axis_index("s") * nc + lax.axis_index("c")
        base = wid * rows_per_w

        for b in range(nbuf_in):
            pltpu.async_copy(
                features_hbm.at[pl.ds(base + b * chunk, chunk)],
                in_v.at[b], sem_in[b])

        tab_copy = pltpu.async_copy(comb_hbm, comb_v, sem_tab)
        pltpu.sync_copy(labels_hbm.at[pl.ds(base, rows_per_w)], lab_v)

        def binify(k, _):
            lv = lab_v[pl.ds(k * _LANE, _LANE)]
            b = ((lv - _MIN_VALUE) * (1.0 / _BIN_WIDTH)).astype(jnp.int32)
            b = jnp.minimum(b, _NBUCKETS - 1) << 7
            bins_v[pl.ds(k * _LANE, _LANE)] = b
            return _

        lax.fori_loop(0, rows_per_w // _LANE, binify, 0)
        tab_copy.wait()

        hi_mask = jnp.int32(-65536)

        def do_chunk(c, bi, bo):
            row0 = base + c * chunk
            pltpu.make_async_copy(
                features_hbm.at[pl.ds(row0, chunk)], in_v.at[bi],
                sem_in[bi]).wait()
            @pl.when(c >= nbuf_out)
            def _():
                pltpu.make_async_copy(
                    out_v.at[bo], out_hbm.at[pl.ds(row0, chunk)],
                    sem_out[bo]).wait()

            def row_body(g, bvec):
                bvec_n = bins_v[pl.ds(c * chunk + (g + 1) * _LANE, _LANE)]
                for l0 in range(0, _LANE, 2):
                    vals = []
                    for l in (l0, l0 + 1):
                        i = g * _LANE + l
                        boff = bvec[l]
                        for j in range(_NGROUPS):
                            x = in_v[bi, i, pl.ds(j * _LANE, _LANE)]
                            w = comb_v[pl.ds(boff + j * _LANE, _LANE)]
                            s = lax.bitcast_convert_type(w << 16, jnp.float32)
                            t = lax.bitcast_convert_type(w & hi_mask,
                                                         jnp.float32)
                            vals.append((i, j, x * s + t))
                    for i, j, r in vals:
                        out_v[bo, i, pl.ds(j * _LANE, _LANE)] = r
                return bvec_n

            bvec0 = bins_v[pl.ds(c * chunk, _LANE)]
            lax.fori_loop(0, chunk // _LANE, row_body, bvec0)

            pltpu.async_copy(
                out_v.at[bo], out_hbm.at[pl.ds(row0, chunk)], sem_out[bo])

            @pl.when(c + nbuf_in < nchunks)
            def _():
                pltpu.async_copy(
                    features_hbm.at[pl.ds(row0 + nbuf_in * chunk, chunk)],
                    in_v.at[bi], sem_in[bi])

        def outer(cc, _):
            for b in range(nbuf_in):
                c = cc * nbuf_in + b
                do_chunk(c, b, b % nbuf_out)
            return _

        lax.fori_loop(0, nchunks // nbuf_in, outer, 0)

        for b in range(nbuf_out):
            c = nchunks - nbuf_out + b
            pltpu.make_async_copy(
                out_v.at[(nchunks - nbuf_out + b) % nbuf_out],
                out_hbm.at[pl.ds(base + c * chunk, chunk)],
                sem_out[(nchunks - nbuf_out + b) % nbuf_out]).wait()

    return sc_kernel


_sc_kernel = None


def kernel(features, labels, running_mean_last_epoch, running_var_last_epoch,
           smoothed_mean_last_epoch, smoothed_var_last_epoch, epoch):
    global _sc_kernel
    if _sc_kernel is None:
        _sc_kernel = _make_sc_kernel()
    epoch_arr = jnp.asarray(epoch, dtype=jnp.int32).reshape((1,))
    comb = _prep_tables(
        running_mean_last_epoch, running_var_last_epoch,
        smoothed_mean_last_epoch, smoothed_var_last_epoch, epoch_arr)
    labels_flat = labels.reshape((_N,))
    return _sc_kernel(features, labels_flat, comb.reshape((-1,)))
